# Initial kernel scaffold; baseline (speedup 1.0000x reference)
import jax
import jax.numpy as jnp
from jax.experimental import pallas as pl


def kernel(inputs, targets):
    raise NotImplementedError("write your pallas kernel here")


# speedup vs baseline: 14.5808x; 14.5808x over previous
import jax
import jax.numpy as jnp
from jax.experimental import pallas as pl
from jax.experimental.pallas import tpu as pltpu

N = 4096
D = 256
NUM_LABELS = 64


def _tc_body(x_ref, t_ref, out_ref):
    x = x_ref[...]
    t = t_ref[...]
    lab = jax.lax.broadcasted_iota(jnp.int32, (N, NUM_LABELS), 1)
    onehot = (t == lab).astype(jnp.float32)
    sums = jax.lax.dot_general(
        onehot, x, (((0,), (0,)), ((), ())),
        preferred_element_type=jnp.float32)
    counts = jnp.sum(onehot, axis=0)
    centers = sums / jnp.maximum(counts, 1.0)[:, None]
    c_rows = jax.lax.dot_general(
        onehot, centers, (((1,), (0,)), ((), ())),
        preferred_element_type=jnp.float32)
    d2 = jnp.sum((x - c_rows) ** 2, axis=1)
    loss = jnp.sum(jnp.sqrt(d2)) * (1.0 / N)
    out_ref[...] = loss.reshape(1, 1)


def kernel(inputs, targets):
    out = pl.pallas_call(
        _tc_body,
        out_shape=jax.ShapeDtypeStruct((1, 1), jnp.float32),
    )(inputs, targets.reshape(N, 1))
    return out[0, 0]
